# baseline (speedup 1.0000x reference)
import jax
import jax.numpy as jnp
from jax.experimental import pallas as pl
from jax.experimental.pallas import tpu as pltpu

_TOKENS = 16384
_HIDDEN = 4096
_EXPERTS = 64
_BT = 512


def _gate_block(thr_ref, hs_ref, sm_ref, rw_ref, scores_ref, k_ref, mask_ref):
    hs = hs_ref[...]
    sm = sm_ref[...]
    rnorm = jnp.sqrt(jnp.sum(hs * hs, axis=1, keepdims=True))
    cnorm = jnp.sqrt(jnp.sum(sm * sm, axis=0, keepdims=True))
    hn = hs * (1.0 / jnp.maximum(rnorm, 1e-12))
    sn = sm * (1.0 / jnp.maximum(cnorm, 1e-12))
    scores = jax.lax.dot_general(
        hn, sn, (((1,), (0,)), ((), ())),
        preferred_element_type=jnp.float32)
    thr = thr_ref[0]
    mask = scores > thr
    k_ref[...] = jnp.sum(mask.astype(jnp.int32), axis=1)
    masked = jnp.where(mask, scores, jnp.float32(-1e9))
    m = jnp.max(masked, axis=1, keepdims=True)
    e = jnp.exp(masked - m)
    rw_ref[...] = e / jnp.sum(e, axis=1, keepdims=True)
    scores_ref[...] = scores
    mask_ref[...] = mask


def kernel(hidden_states, sim_matrix, threshold):
    grid = (_TOKENS // _BT,)
    out = pl.pallas_call(
        _gate_block,
        grid=grid,
        in_specs=[
            pl.BlockSpec(memory_space=pltpu.SMEM),
            pl.BlockSpec((_BT, _HIDDEN), lambda i: (i, 0)),
            pl.BlockSpec((_HIDDEN, _EXPERTS), lambda i: (0, 0)),
        ],
        out_specs=[
            pl.BlockSpec((_BT, _EXPERTS), lambda i: (i, 0)),
            pl.BlockSpec((_BT, _EXPERTS), lambda i: (i, 0)),
            pl.BlockSpec((_BT,), lambda i: (i,)),
            pl.BlockSpec((_BT, _EXPERTS), lambda i: (i, 0)),
        ],
        out_shape=[
            jax.ShapeDtypeStruct((_TOKENS, _EXPERTS), jnp.float32),
            jax.ShapeDtypeStruct((_TOKENS, _EXPERTS), jnp.float32),
            jax.ShapeDtypeStruct((_TOKENS,), jnp.int32),
            jax.ShapeDtypeStruct((_TOKENS, _EXPERTS), jnp.bool_),
        ],
        compiler_params=pltpu.CompilerParams(
            dimension_semantics=("parallel",),
        ),
    )(threshold, hidden_states, sim_matrix)
    routing_weights, scores, k_per_token, activated_mask = out
    return routing_weights, scores, k_per_token, activated_mask


# speedup vs baseline: 1.1524x; 1.1524x over previous
import jax
import jax.numpy as jnp
from jax.experimental import pallas as pl
from jax.experimental.pallas import tpu as pltpu

_TOKENS = 16384
_HIDDEN = 4096
_EXPERTS = 64
_BT = 1024


def _gate_block(thr_ref, hs_ref, sm_ref, rw_ref, scores_ref, k_ref, mask_ref):
    hs = hs_ref[...]
    sm = sm_ref[...]
    rnorm = jnp.sqrt(jnp.sum(hs * hs, axis=1, keepdims=True))
    cnorm = jnp.sqrt(jnp.sum(sm * sm, axis=0, keepdims=True))
    hn = hs * (1.0 / jnp.maximum(rnorm, 1e-12))
    sn = sm * (1.0 / jnp.maximum(cnorm, 1e-12))
    scores = jax.lax.dot_general(
        hn, sn, (((1,), (0,)), ((), ())),
        preferred_element_type=jnp.float32)
    thr = thr_ref[0]
    mask = scores > thr
    k_ref[...] = jnp.sum(mask.astype(jnp.int32), axis=1)
    masked = jnp.where(mask, scores, jnp.float32(-1e9))
    m = jnp.max(masked, axis=1, keepdims=True)
    e = jnp.exp(masked - m)
    rw_ref[...] = e / jnp.sum(e, axis=1, keepdims=True)
    scores_ref[...] = scores
    mask_ref[...] = mask


def kernel(hidden_states, sim_matrix, threshold):
    grid = (_TOKENS // _BT,)
    out = pl.pallas_call(
        _gate_block,
        grid=grid,
        in_specs=[
            pl.BlockSpec(memory_space=pltpu.SMEM),
            pl.BlockSpec((_BT, _HIDDEN), lambda i: (i, 0)),
            pl.BlockSpec((_HIDDEN, _EXPERTS), lambda i: (0, 0)),
        ],
        out_specs=[
            pl.BlockSpec((_BT, _EXPERTS), lambda i: (i, 0)),
            pl.BlockSpec((_BT, _EXPERTS), lambda i: (i, 0)),
            pl.BlockSpec((_BT,), lambda i: (i,)),
            pl.BlockSpec((_BT, _EXPERTS), lambda i: (i, 0)),
        ],
        out_shape=[
            jax.ShapeDtypeStruct((_TOKENS, _EXPERTS), jnp.float32),
            jax.ShapeDtypeStruct((_TOKENS, _EXPERTS), jnp.float32),
            jax.ShapeDtypeStruct((_TOKENS,), jnp.int32),
            jax.ShapeDtypeStruct((_TOKENS, _EXPERTS), jnp.bool_),
        ],
        compiler_params=pltpu.CompilerParams(
            dimension_semantics=("parallel",),
        ),
    )(threshold, hidden_states, sim_matrix)
    routing_weights, scores, k_per_token, activated_mask = out
    return routing_weights, scores, k_per_token, activated_mask


# speedup vs baseline: 1.1752x; 1.0198x over previous
import jax
import jax.numpy as jnp
from jax.experimental import pallas as pl
from jax.experimental.pallas import tpu as pltpu

_TOKENS = 16384
_HIDDEN = 4096
_EXPERTS = 64
_BT = 1024


def _gate_block(thr_ref, hs_ref, sm_ref, rw_ref, scores_ref, k_ref, mask_ref,
                sn_ref):
    @pl.when(pl.program_id(0) == 0)
    def _():
        sm = sm_ref[...]
        cnorm = jnp.sqrt(jnp.sum(sm * sm, axis=0, keepdims=True))
        sn_ref[...] = sm * (1.0 / jnp.maximum(cnorm, 1e-12))

    hs = hs_ref[...]
    rnorm = jnp.sqrt(jnp.sum(hs * hs, axis=1, keepdims=True))
    hn = hs * (1.0 / jnp.maximum(rnorm, 1e-12))
    scores = jax.lax.dot_general(
        hn, sn_ref[...], (((1,), (0,)), ((), ())),
        preferred_element_type=jnp.float32)
    thr = thr_ref[0]
    mask = scores > thr
    k = jnp.sum(mask.astype(jnp.int32), axis=1)
    k_ref[...] = k
    e = jnp.where(mask, jnp.exp(scores), 0.0)
    denom = jnp.sum(e, axis=1, keepdims=True)
    rw = e / jnp.maximum(denom, 1e-30)
    rw_ref[...] = jnp.where((k == 0)[:, None], 1.0 / _EXPERTS, rw)
    scores_ref[...] = scores
    mask_ref[...] = mask


def kernel(hidden_states, sim_matrix, threshold):
    grid = (_TOKENS // _BT,)
    out = pl.pallas_call(
        _gate_block,
        grid=grid,
        in_specs=[
            pl.BlockSpec(memory_space=pltpu.SMEM),
            pl.BlockSpec((_BT, _HIDDEN), lambda i: (i, 0)),
            pl.BlockSpec((_HIDDEN, _EXPERTS), lambda i: (0, 0)),
        ],
        out_specs=[
            pl.BlockSpec((_BT, _EXPERTS), lambda i: (i, 0)),
            pl.BlockSpec((_BT, _EXPERTS), lambda i: (i, 0)),
            pl.BlockSpec((_BT,), lambda i: (i,)),
            pl.BlockSpec((_BT, _EXPERTS), lambda i: (i, 0)),
        ],
        out_shape=[
            jax.ShapeDtypeStruct((_TOKENS, _EXPERTS), jnp.float32),
            jax.ShapeDtypeStruct((_TOKENS, _EXPERTS), jnp.float32),
            jax.ShapeDtypeStruct((_TOKENS,), jnp.int32),
            jax.ShapeDtypeStruct((_TOKENS, _EXPERTS), jnp.bool_),
        ],
        scratch_shapes=[pltpu.VMEM((_HIDDEN, _EXPERTS), jnp.float32)],
        compiler_params=pltpu.CompilerParams(
            dimension_semantics=("arbitrary",),
        ),
    )(threshold, hidden_states, sim_matrix)
    routing_weights, scores, k_per_token, activated_mask = out
    return routing_weights, scores, k_per_token, activated_mask
